# baseline (speedup 1.0000x reference)
import functools

import jax
import jax.numpy as jnp
from jax.experimental import pallas as pl
from jax.experimental.pallas import tpu as pltpu

D_MODEL = 1024
B_TOTAL = 4 * 4096
RB = 512
NB = B_TOTAL // RB


def _tc_body(x_ref, w_ref, out_ref):
    xv = x_ref[...].astype(jnp.float32).reshape(RB, 1)
    ang = xv * w_ref[0:1, :] + w_ref[1:2, :]
    out_ref[...] = jnp.sin(ang)


def _pe_compute(x2, wp):
    return pl.pallas_call(
        _tc_body,
        grid=(NB,),
        in_specs=[
            pl.BlockSpec((1, 1, RB), lambda i: (i, 0, 0)),
            pl.BlockSpec((2, D_MODEL), lambda i: (0, 0)),
        ],
        out_specs=pl.BlockSpec((RB, D_MODEL), lambda i: (i, 0)),
        out_shape=jax.ShapeDtypeStruct((B_TOTAL, D_MODEL), jnp.float32),
    )(x2, wp)


def kernel(x, pe):
    div_term = jnp.exp(
        jnp.arange(0, D_MODEL, 2, dtype=jnp.float32)
        * -(jnp.log(jnp.float32(10000.0)) / D_MODEL))
    wfull = jnp.repeat(div_term, 2)
    phase = jnp.tile(jnp.array([0.0, jnp.pi / 2], dtype=jnp.float32),
                     D_MODEL // 2)
    wp = jnp.stack([wfull, phase])
    x2 = x.reshape(NB, 1, RB).astype(jnp.int32)
    out = _pe_compute(x2, wp)
    return out.reshape(x.shape + (D_MODEL,))


# speedup vs baseline: 3.3166x; 3.3166x over previous
import functools

import jax
import jax.numpy as jnp
from jax import lax
from jax.experimental import pallas as pl
from jax.experimental.pallas import tpu as pltpu
from jax.experimental.pallas import tpu_sc as plsc

D_MODEL = 1024
B_TOTAL = 4 * 4096
NUM_CORES = 2
NUM_SUBCORES = 16
NW = NUM_CORES * NUM_SUBCORES
B_PER_W = B_TOTAL // NW
CHUNK = 32
NCHUNK = B_PER_W // CHUNK
NBUF = 3


def _pe_gather(x_grouped, pe):
    mesh = plsc.VectorSubcoreMesh(core_axis_name="c", subcore_axis_name="s")

    @functools.partial(
        pl.kernel,
        mesh=mesh,
        out_type=jax.ShapeDtypeStruct((B_TOTAL, D_MODEL), jnp.float32),
        scratch_types=[
            pltpu.VMEM((NCHUNK, CHUNK), jnp.int32),
        ]
        + [pltpu.VMEM((CHUNK, D_MODEL), jnp.float32) for _ in range(NBUF)]
        + [pltpu.SemaphoreType.DMA for _ in range(2 * NBUF)],
    )
    def k(idx_hbm, table_hbm, out_hbm, idx_v, *scratch):
        bufs = scratch[:NBUF]
        gsems = scratch[NBUF:2 * NBUF]
        osems = scratch[2 * NBUF:]
        wid = lax.axis_index("s") * NUM_CORES + lax.axis_index("c")
        base = wid * B_PER_W
        pltpu.sync_copy(idx_hbm.at[wid], idx_v)
        gcp = [None] * NBUF
        ocp = [None] * NBUF
        for g in range(NBUF):
            gcp[g] = pltpu.async_copy(
                table_hbm.at[idx_v.at[g]], bufs[g], gsems[g])
        for c in range(NCHUNK):
            b = c % NBUF
            gcp[b].wait()
            ocp[b] = pltpu.async_copy(
                bufs[b], out_hbm.at[pl.ds(base + c * CHUNK, CHUNK)],
                osems[b])
            g = c + NBUF
            if g < NCHUNK:
                ocp[b].wait()
                gcp[b] = pltpu.async_copy(
                    table_hbm.at[idx_v.at[g]], bufs[b], gsems[b])
        for c in range(NCHUNK - NBUF, NCHUNK):
            if c >= 0:
                ocp[c % NBUF].wait()

    return k(x_grouped, pe)


def kernel(x, pe):
    x_grouped = x.reshape(NW, NCHUNK, CHUNK).astype(jnp.int32)
    out = _pe_gather(x_grouped, pe.astype(jnp.float32))
    return out.reshape(x.shape + (D_MODEL,))
